# Initial kernel scaffold; baseline (speedup 1.0000x reference)
import functools

import jax
import jax.numpy as jnp
from jax import lax
from jax.experimental import pallas as pl
from jax.experimental.pallas import tpu as pltpu
from jax.experimental.pallas import tpu_sc as plsc

N_NODES = 50000
WALK = 16
HID = 64

NSUB = 16
SLICE = 3136
NPAD = NSUB * SLICE
CH = 2000


def _pack_body(row_ref, col_ref, out_ref):
    out_ref[...] = row_ref[...] | lax.shift_left(col_ref[...], 16)


def _mlp_body(r_ref, w1_ref, b1_ref, w2_ref, b2_ref, o_ref):
    x = r_ref[...]
    h = lax.dot_general(x, w1_ref[...], (((0,), (0,)), ((), ())),
                        preferred_element_type=jnp.float32,
                        precision=lax.Precision.HIGHEST)
    h = jnp.maximum(h + b1_ref[...], 0.0)
    o = lax.dot_general(h, w2_ref[...], (((1,), (0,)), ((), ())),
                        preferred_element_type=jnp.float32,
                        precision=lax.Precision.HIGHEST)
    o_ref[...] = o + b2_ref[...]


def _sc_walk(pairs):
    E = pairs.shape[0]
    ET = E // NSUB
    NCH = ET // CH
    assert ET % CH == 0 and NCH % 2 == 0 and CH % 80 == 0

    mesh = plsc.VectorSubcoreMesh(core_axis_name="c", subcore_axis_name="s")

    @functools.partial(
        pl.kernel,
        mesh=mesh,
        out_type=(
            jax.ShapeDtypeStruct((WALK, NPAD), jnp.float32),
            jax.ShapeDtypeStruct((2, NSUB, NPAD), jnp.float32),
            jax.ShapeDtypeStruct((2, NPAD), jnp.float32),
        ),
        scratch_types=[
            pltpu.VMEM((NPAD,), jnp.float32),
            pltpu.VMEM((NPAD,), jnp.float32),
            pltpu.VMEM((2, CH), jnp.int32),
            pltpu.VMEM((4, SLICE), jnp.float32),
            pltpu.VMEM((SLICE,), jnp.float32),
            pltpu.VMEM((SLICE,), jnp.float32),
            pltpu.VMEM((SLICE,), jnp.float32),
            pltpu.SemaphoreType.DMA,
            pltpu.SemaphoreType.DMA,
        ],
    )
    def walk(pairs_hbm, rwpe_hbm, part_hbm, q_hbm,
             q_v, acc_v, pairbuf, stage, pslice, qs_v, dinv_v, sem0, sem1):
        cid = lax.axis_index("c")
        w = lax.axis_index("s")
        base = w * SLICE
        ebase = w * ET
        zeros16 = jnp.zeros((16,), jnp.float32)
        ones16 = jnp.ones((16,), jnp.float32)
        iota16 = lax.iota(jnp.int32, 16)

        def zero_acc():
            def body(i, _):
                for u in range(8):
                    acc_v[pl.ds((i * 8 + u) * 16, 16)] = zeros16
                return 0
            lax.fori_loop(0, NPAD // 128, body, 0)

        def chunk_start(cidx, slot, sem):
            pltpu.make_async_copy(
                pairs_hbm.at[pl.ds(ebase + cidx * CH, CH)],
                pairbuf.at[slot], sem).start()

        def chunk_wait(cidx, slot, sem):
            pltpu.make_async_copy(
                pairs_hbm.at[pl.ds(ebase + cidx * CH, CH)],
                pairbuf.at[slot], sem).wait()

        def process(slot, use_gather):
            def body(j, _):
                for u in range(5):
                    off = (j * 5 + u) * 16
                    pair = pairbuf[slot, pl.ds(off, 16)]
                    row = jnp.bitwise_and(pair, 0xFFFF)
                    if use_gather:
                        col = lax.shift_right_logical(pair, 16)
                        vals = plsc.load_gather(q_v, [row])
                        plsc.addupdate_scatter(acc_v, [col], vals)
                    else:
                        plsc.addupdate_scatter(acc_v, [row], ones16)
                return 0
            lax.fori_loop(0, CH // 80, body, 0)

        def edge_pass(use_gather):
            zero_acc()
            chunk_start(0, 0, sem0)
            chunk_start(1, 1, sem1)

            def cbody(c, _):
                c0 = 2 * c
                chunk_wait(c0, 0, sem0)
                process(0, use_gather)
                chunk_start(jnp.minimum(c0 + 2, NCH - 1), 0, sem0)
                chunk_wait(c0 + 1, 1, sem1)
                process(1, use_gather)
                chunk_start(jnp.minimum(c0 + 3, NCH - 1), 1, sem1)
                return 0
            lax.fori_loop(0, NCH // 2, cbody, 0)
            chunk_wait(NCH - 1, 0, sem0)
            chunk_wait(NCH - 1, 1, sem1)
            pltpu.sync_copy(acc_v, part_hbm.at[cid, w])
            plsc.subcore_barrier()

        def combine():
            for g in range(4):
                pltpu.sync_copy(
                    part_hbm.at[cid, pl.ds(g * 4, 4), pl.ds(base, SLICE)],
                    stage)

                def body(i, _):
                    for u in range(4):
                        ds = pl.ds((i * 4 + u) * 16, 16)
                        s = ((stage[0, ds] + stage[1, ds])
                             + (stage[2, ds] + stage[3, ds]))
                        if g > 0:
                            s = s + pslice[ds]
                        pslice[ds] = s
                    return 0
                lax.fori_loop(0, SLICE // 64, body, 0)

        edge_pass(use_gather=False)
        combine()

        def dfin(i, _):
            for u in range(4):
                idx0 = (i * 4 + u) * 16
                ds = pl.ds(idx0, 16)
                deg = pslice[ds] + 1.0
                gid = base + idx0 + iota16
                dinv = jnp.where(gid < N_NODES,
                                 1.0 / jnp.maximum(deg, 1e-8), 0.0)
                dinv_v[ds] = dinv
            return 0
        lax.fori_loop(0, SLICE // 64, dfin, 0)
        pltpu.sync_copy(dinv_v, q_hbm.at[cid, pl.ds(base, SLICE)])
        plsc.subcore_barrier()
        pltpu.sync_copy(q_hbm.at[cid], q_v)

        def step(k, _):
            edge_pass(use_gather=True)
            combine()

            def sfin(i, _):
                for u in range(4):
                    idx0 = (i * 4 + u) * 16
                    ds = pl.ds(idx0, 16)
                    pv = pslice[ds] + q_v[pl.ds(base + idx0, 16)]
                    pslice[ds] = pv
                    qs_v[ds] = pv * dinv_v[ds]
                return 0
            lax.fori_loop(0, SLICE // 64, sfin, 0)

            @pl.when(cid == 0)
            def _():
                pltpu.sync_copy(pslice, rwpe_hbm.at[k, pl.ds(base, SLICE)])
            pltpu.sync_copy(qs_v, q_hbm.at[cid, pl.ds(base, SLICE)])
            plsc.subcore_barrier()
            pltpu.sync_copy(q_hbm.at[cid], q_v)
            return 0
        lax.fori_loop(0, WALK, step, 0)

    rwpe_T, _, _ = walk(pairs)
    return rwpe_T


def kernel(edge_index, num_nodes, W1, b1, W2, b2):
    E = edge_index.shape[1]
    row2d = edge_index[0].reshape(E // 128, 128)
    col2d = edge_index[1].reshape(E // 128, 128)

    nrow = E // 128
    pack_blk = 1250
    pairs2d = pl.pallas_call(
        _pack_body,
        out_shape=jax.ShapeDtypeStruct((nrow, 128), jnp.int32),
        grid=(nrow // pack_blk,),
        in_specs=[
            pl.BlockSpec((pack_blk, 128), lambda i: (i, 0)),
            pl.BlockSpec((pack_blk, 128), lambda i: (i, 0)),
        ],
        out_specs=pl.BlockSpec((pack_blk, 128), lambda i: (i, 0)),
    )(row2d, col2d)
    pairs = pairs2d.reshape(E)

    rwpe_T = _sc_walk(pairs)

    bn = NPAD // 8
    out_full = pl.pallas_call(
        _mlp_body,
        out_shape=jax.ShapeDtypeStruct((NPAD, HID), jnp.float32),
        grid=(8,),
        in_specs=[
            pl.BlockSpec((WALK, bn), lambda i: (0, i)),
            pl.BlockSpec((WALK, HID), lambda i: (0, 0)),
            pl.BlockSpec((1, HID), lambda i: (0, 0)),
            pl.BlockSpec((HID, HID), lambda i: (0, 0)),
            pl.BlockSpec((1, HID), lambda i: (0, 0)),
        ],
        out_specs=pl.BlockSpec((bn, HID), lambda i: (i, 0)),
    )(rwpe_T, W1, b1.reshape(1, HID), W2, b2.reshape(1, HID))

    return out_full[:N_NODES]


# speedup vs baseline: 127.9485x; 127.9485x over previous
import functools

import jax
import jax.numpy as jnp
from jax import lax
from jax.experimental import pallas as pl
from jax.experimental.pallas import tpu as pltpu
from jax.experimental.pallas import tpu_sc as plsc

N_NODES = 50000
WALK = 16
HID = 64

NSUB = 16
SLICE = 3136
NPAD = NSUB * SLICE
CH = 2000


def _pack_body(row_ref, col_ref, out_ref):
    out_ref[...] = row_ref[...] | lax.shift_left(col_ref[...], 16)


def _mlp_body(r_ref, w1_ref, b1_ref, w2_ref, b2_ref, o_ref):
    x = r_ref[...]
    h = lax.dot_general(x, w1_ref[...], (((0,), (0,)), ((), ())),
                        preferred_element_type=jnp.float32,
                        precision=lax.Precision.HIGHEST)
    h = jnp.maximum(h + b1_ref[...], 0.0)
    o = lax.dot_general(h, w2_ref[...], (((1,), (0,)), ((), ())),
                        preferred_element_type=jnp.float32,
                        precision=lax.Precision.HIGHEST)
    o_ref[...] = o + b2_ref[...]


def _sc_walk(pairs):
    E = pairs.shape[0]
    ET = E // NSUB
    NCH = ET // CH
    assert ET % CH == 0 and NCH % 2 == 0 and CH % 80 == 0

    mesh = plsc.VectorSubcoreMesh(core_axis_name="c", subcore_axis_name="s")

    @functools.partial(
        pl.kernel,
        mesh=mesh,
        compiler_params=pltpu.CompilerParams(use_tc_tiling_on_sc=False,
                                             needs_layout_passes=False),
        out_type=(
            jax.ShapeDtypeStruct((WALK, NPAD), jnp.float32),
            jax.ShapeDtypeStruct((2, NSUB, NPAD), jnp.float32),
            jax.ShapeDtypeStruct((2, NPAD), jnp.float32),
        ),
        scratch_types=[
            pltpu.VMEM((NPAD,), jnp.float32),
            pltpu.VMEM((NPAD,), jnp.float32),
            pltpu.VMEM((2, CH), jnp.int32),
            pltpu.VMEM((4, SLICE), jnp.float32),
            pltpu.VMEM((SLICE,), jnp.float32),
            pltpu.VMEM((SLICE,), jnp.float32),
            pltpu.VMEM((SLICE,), jnp.float32),
            pltpu.SemaphoreType.DMA,
            pltpu.SemaphoreType.DMA,
        ],
    )
    def walk(pairs_hbm, rwpe_hbm, part_hbm, q_hbm,
             q_v, acc_v, pairbuf, stage, pslice, qs_v, dinv_v, sem0, sem1):
        cid = lax.axis_index("c")
        w = lax.axis_index("s")
        base = w * SLICE
        ebase = w * ET
        zeros16 = jnp.zeros((16,), jnp.float32)
        ones16 = jnp.ones((16,), jnp.float32)
        iota16 = lax.iota(jnp.int32, 16)

        def zero_acc():
            def body(i, _):
                for u in range(8):
                    acc_v[pl.ds((i * 8 + u) * 16, 16)] = zeros16
                return 0
            lax.fori_loop(0, NPAD // 128, body, 0)

        def chunk_start(cidx, slot, sem):
            pltpu.make_async_copy(
                pairs_hbm.at[pl.ds(ebase + cidx * CH, CH)],
                pairbuf.at[slot], sem).start()

        def chunk_wait(cidx, slot, sem):
            pltpu.make_async_copy(
                pairs_hbm.at[pl.ds(ebase + cidx * CH, CH)],
                pairbuf.at[slot], sem).wait()

        def process(slot, use_gather):
            def body(j, _):
                for u in range(5):
                    off = (j * 5 + u) * 16
                    pair = pairbuf[slot, pl.ds(off, 16)]
                    row = jnp.bitwise_and(pair, 0xFFFF)
                    if use_gather:
                        col = lax.shift_right_logical(pair, 16)
                        vals = plsc.load_gather(q_v, [row])
                        plsc.addupdate_scatter(acc_v, [col], vals)
                    else:
                        plsc.addupdate_scatter(acc_v, [row], ones16)
                return 0
            lax.fori_loop(0, CH // 80, body, 0)

        def edge_pass(use_gather):
            zero_acc()
            chunk_start(0, 0, sem0)
            chunk_start(1, 1, sem1)

            def cbody(c, _):
                c0 = 2 * c
                chunk_wait(c0, 0, sem0)
                process(0, use_gather)
                chunk_start(jnp.minimum(c0 + 2, NCH - 1), 0, sem0)
                chunk_wait(c0 + 1, 1, sem1)
                process(1, use_gather)
                chunk_start(jnp.minimum(c0 + 3, NCH - 1), 1, sem1)
                return 0
            lax.fori_loop(0, NCH // 2, cbody, 0)
            chunk_wait(NCH - 1, 0, sem0)
            chunk_wait(NCH - 1, 1, sem1)
            pltpu.sync_copy(acc_v, part_hbm.at[cid, w])
            plsc.subcore_barrier()

        def combine():
            for g in range(4):
                pltpu.sync_copy(
                    part_hbm.at[cid, pl.ds(g * 4, 4), pl.ds(base, SLICE)],
                    stage)

                def body(i, _):
                    for u in range(4):
                        ds = pl.ds((i * 4 + u) * 16, 16)
                        s = ((stage[0, ds] + stage[1, ds])
                             + (stage[2, ds] + stage[3, ds]))
                        if g > 0:
                            s = s + pslice[ds]
                        pslice[ds] = s
                    return 0
                lax.fori_loop(0, SLICE // 64, body, 0)

        edge_pass(use_gather=False)
        combine()

        def dfin(i, _):
            for u in range(4):
                idx0 = (i * 4 + u) * 16
                ds = pl.ds(idx0, 16)
                deg = pslice[ds] + 1.0
                gid = base + idx0 + iota16
                dinv = jnp.where(gid < N_NODES,
                                 1.0 / jnp.maximum(deg, 1e-8), 0.0)
                dinv_v[ds] = dinv
            return 0
        lax.fori_loop(0, SLICE // 64, dfin, 0)
        pltpu.sync_copy(dinv_v, q_hbm.at[cid, pl.ds(base, SLICE)])
        plsc.subcore_barrier()
        pltpu.sync_copy(q_hbm.at[cid], q_v)

        def step(k, _):
            edge_pass(use_gather=True)
            combine()

            def sfin(i, _):
                for u in range(4):
                    idx0 = (i * 4 + u) * 16
                    ds = pl.ds(idx0, 16)
                    pv = pslice[ds] + q_v[pl.ds(base + idx0, 16)]
                    pslice[ds] = pv
                    qs_v[ds] = pv * dinv_v[ds]
                return 0
            lax.fori_loop(0, SLICE // 64, sfin, 0)

            @pl.when(cid == 0)
            def _():
                pltpu.sync_copy(pslice, rwpe_hbm.at[k, pl.ds(base, SLICE)])
            pltpu.sync_copy(qs_v, q_hbm.at[cid, pl.ds(base, SLICE)])
            plsc.subcore_barrier()
            pltpu.sync_copy(q_hbm.at[cid], q_v)
            return 0
        lax.fori_loop(0, WALK, step, 0)

    rwpe_T, _, _ = walk(pairs)
    return rwpe_T


def kernel(edge_index, num_nodes, W1, b1, W2, b2):
    E = edge_index.shape[1]
    row2d = edge_index[0].reshape(E // 128, 128)
    col2d = edge_index[1].reshape(E // 128, 128)

    nrow = E // 128
    pairs2d = pl.pallas_call(
        _pack_body,
        out_shape=jax.ShapeDtypeStruct((nrow, 128), jnp.int32),
    )(row2d, col2d)
    pairs = pairs2d.reshape(E)

    rwpe_T = _sc_walk(pairs)

    bn = NPAD // 8
    out_full = pl.pallas_call(
        _mlp_body,
        out_shape=jax.ShapeDtypeStruct((NPAD, HID), jnp.float32),
        grid=(8,),
        in_specs=[
            pl.BlockSpec((WALK, bn), lambda i: (0, i)),
            pl.BlockSpec((WALK, HID), lambda i: (0, 0)),
            pl.BlockSpec((1, HID), lambda i: (0, 0)),
            pl.BlockSpec((HID, HID), lambda i: (0, 0)),
            pl.BlockSpec((1, HID), lambda i: (0, 0)),
        ],
        out_specs=pl.BlockSpec((bn, HID), lambda i: (i, 0)),
    )(rwpe_T, W1, b1.reshape(1, HID), W2, b2.reshape(1, HID))

    return out_full[:N_NODES]
